# baseline (speedup 1.0000x reference)
import functools

import jax
import jax.numpy as jnp
from jax.experimental import pallas as pl

ALPHA = 1.0
NN_SIZE = 5
RADIUS = 0.07
H = 0.03
EPS = 1e-12

B = 4
N = 2048
D = 3
TILE = 1024
BIG = 3.0e38


def _loss_kernel(pred_ref, rgt_ref, rpp_ref, emd_ref, rep_ref):
    b = pl.program_id(0)
    t = pl.program_id(1)

    p = pred_ref[0]
    rgt = rgt_ref[0]
    rpp = rpp_ref[0]

    a2 = (p[:, 0:1] * p[:, 0:1] + p[:, 1:2] * p[:, 1:2]
          + p[:, 2:3] * p[:, 2:3])

    def rowmin(x):
        return jnp.min(x, axis=1, keepdims=True)

    def tiles(rhs):
        mm = jax.lax.dot_general(p, rhs[0:3, :], (((1,), (0,)), ((), ())),
                                 preferred_element_type=jnp.float32)
        mm_hi = jax.lax.dot_general(p, rhs[0:3, :], (((1,), (0,)), ((), ())),
                                    preferred_element_type=jnp.float32,
                                    precision=jax.lax.Precision.HIGHEST)
        tt = a2 + rhs[3:4, :]
        d = tt - 2.0 * mm
        v = tt - 2.0 * mm_hi
        return d, v

    d_gt, v_gt = tiles(rgt)
    m = rowmin(d_gt)
    val = rowmin(jnp.where(d_gt == m, v_gt, BIG))
    emd_val = jnp.sum(val)

    d_pp, v_pp = tiles(rpp)
    prev = rowmin(d_pp)
    rep_val = jnp.zeros((), dtype=jnp.float32)
    for _ in range(NN_SIZE - 1):
        cur = rowmin(jnp.where(d_pp > prev, d_pp, BIG))
        vk = rowmin(jnp.where(d_pp == cur, v_pp, BIG))
        d2 = jnp.maximum(vk, EPS)
        dist = jnp.sqrt(d2)
        w = jnp.exp(-d2 / (H * H))
        term = jnp.where(cur < BIG, (RADIUS - dist) * w, 0.0)
        rep_val = rep_val + jnp.sum(term)
        prev = cur

    @pl.when(jnp.logical_and(b == 0, t == 0))
    def _():
        emd_ref[...] = jnp.zeros_like(emd_ref)
        rep_ref[...] = jnp.zeros_like(rep_ref)

    emd_ref[pl.ds(b, 1), :] += emd_val.reshape(1, 1)
    rep_ref[...] += rep_val.reshape(1, 1)


@functools.partial(jax.jit, static_argnames=("interpret",))
def kernel(pred, gt, pcd_radius, interpret=False):
    g2 = jnp.sum(gt * gt, axis=2, keepdims=True)
    rhs_gt = jnp.concatenate([gt, g2], axis=2).transpose(0, 2, 1)
    p2 = jnp.sum(pred * pred, axis=2, keepdims=True)
    rhs_pp = jnp.concatenate([pred, p2], axis=2).transpose(0, 2, 1)

    emd_sums, rep_sum = pl.pallas_call(
        _loss_kernel,
        grid=(B, N // TILE),
        in_specs=[
            pl.BlockSpec((1, TILE, D), lambda b, t: (b, t, 0)),
            pl.BlockSpec((1, 4, N), lambda b, t: (b, 0, 0)),
            pl.BlockSpec((1, 4, N), lambda b, t: (b, 0, 0)),
        ],
        out_specs=[
            pl.BlockSpec((B, 1), lambda b, t: (0, 0)),
            pl.BlockSpec((1, 1), lambda b, t: (0, 0)),
        ],
        out_shape=[
            jax.ShapeDtypeStruct((B, 1), jnp.float32),
            jax.ShapeDtypeStruct((1, 1), jnp.float32),
        ],
        interpret=interpret,
    )(pred, rhs_gt, rhs_pp)

    dist2_mean = emd_sums / float(N * D) / pcd_radius
    emd_loss = jnp.mean(dist2_mean) * 100.0
    uniform_loss = rep_sum[0, 0] / float(B * N * (NN_SIZE - 1))
    return (emd_loss, ALPHA * uniform_loss)


# speedup vs baseline: 1.0743x; 1.0743x over previous
import functools

import jax
import jax.numpy as jnp
from jax.experimental import pallas as pl

ALPHA = 1.0
NN_SIZE = 5
RADIUS = 0.07
H = 0.03
EPS = 1e-12

B = 4
N = 2048
D = 3
TILE = 1024
BIG = 3.0e38


def _loss_kernel(paug_ref, rgt_ref, rpp_ref, emd_ref, rep_ref):
    b = pl.program_id(0)
    t = pl.program_id(1)

    pa = paug_ref[0]
    rgt = rgt_ref[0]
    rpp = rpp_ref[0]

    p = pa[:, 0:3]
    a2 = (pa[:, 0:1] * pa[:, 0:1] + pa[:, 1:2] * pa[:, 1:2]
          + pa[:, 2:3] * pa[:, 2:3])

    def rowmin(x):
        return jnp.min(x, axis=1, keepdims=True)

    def tiles(rhs):
        m2ab = jax.lax.dot_general(p, rhs[0:3, :], (((1,), (0,)), ((), ())),
                                   preferred_element_type=jnp.float32)
        d = (a2 + rhs[3:4, :]) + m2ab
        v = jax.lax.dot_general(pa, rhs, (((1,), (0,)), ((), ())),
                                preferred_element_type=jnp.float32,
                                precision=jax.lax.Precision.HIGHEST)
        return d, v

    d_gt, v_gt = tiles(rgt)
    m = rowmin(d_gt)
    val = rowmin(jnp.where(d_gt == m, v_gt, BIG))
    emd_val = jnp.sum(val + a2)

    d_pp, v_pp = tiles(rpp)
    prev = rowmin(d_pp)
    rep_val = jnp.zeros((), dtype=jnp.float32)
    for _ in range(NN_SIZE - 1):
        cur = rowmin(jnp.where(d_pp > prev, d_pp, BIG))
        vk = rowmin(jnp.where(d_pp == cur, v_pp, BIG))
        d2 = jnp.maximum(vk + a2, EPS)
        dist = jnp.sqrt(d2)
        w = jnp.exp(-d2 / (H * H))
        term = jnp.where(cur < BIG, (RADIUS - dist) * w, 0.0)
        rep_val = rep_val + jnp.sum(term)
        prev = cur

    @pl.when(jnp.logical_and(b == 0, t == 0))
    def _():
        emd_ref[...] = jnp.zeros_like(emd_ref)
        rep_ref[...] = jnp.zeros_like(rep_ref)

    emd_ref[pl.ds(b, 1), :] += emd_val.reshape(1, 1)
    rep_ref[...] += rep_val.reshape(1, 1)


@functools.partial(jax.jit, static_argnames=("interpret",))
def kernel(pred, gt, pcd_radius, interpret=False):
    ones = jnp.ones(pred.shape[:2] + (1,), dtype=pred.dtype)
    p_aug = jnp.concatenate([pred, ones], axis=2)
    g2 = jnp.sum(gt * gt, axis=2, keepdims=True)
    rhs_gt = jnp.concatenate([-2.0 * gt, g2], axis=2).transpose(0, 2, 1)
    p2 = jnp.sum(pred * pred, axis=2, keepdims=True)
    rhs_pp = jnp.concatenate([-2.0 * pred, p2], axis=2).transpose(0, 2, 1)

    emd_sums, rep_sum = pl.pallas_call(
        _loss_kernel,
        grid=(B, N // TILE),
        in_specs=[
            pl.BlockSpec((1, TILE, 4), lambda b, t: (b, t, 0)),
            pl.BlockSpec((1, 4, N), lambda b, t: (b, 0, 0)),
            pl.BlockSpec((1, 4, N), lambda b, t: (b, 0, 0)),
        ],
        out_specs=[
            pl.BlockSpec((B, 1), lambda b, t: (0, 0)),
            pl.BlockSpec((1, 1), lambda b, t: (0, 0)),
        ],
        out_shape=[
            jax.ShapeDtypeStruct((B, 1), jnp.float32),
            jax.ShapeDtypeStruct((1, 1), jnp.float32),
        ],
        interpret=interpret,
    )(p_aug, rhs_gt, rhs_pp)

    dist2_mean = emd_sums / float(N * D) / pcd_radius
    emd_loss = jnp.mean(dist2_mean) * 100.0
    uniform_loss = rep_sum[0, 0] / float(B * N * (NN_SIZE - 1))
    return (emd_loss, ALPHA * uniform_loss)


# speedup vs baseline: 1.0743x; 1.0000x over previous
import functools

import jax
import jax.numpy as jnp
from jax.experimental import pallas as pl

ALPHA = 1.0
NN_SIZE = 5
RADIUS = 0.07
H = 0.03
EPS = 1e-12

B = 4
N = 2048
D = 3
TILE = 1024
BIG = 3.0e38


def _loss_kernel(paug_ref, rgt_ref, rpp_ref, emd_ref, rep_ref):
    b = pl.program_id(0)
    t = pl.program_id(1)

    pa = paug_ref[0]
    rgt = rgt_ref[0]
    rpp = rpp_ref[0]

    p = pa[:, 0:3]
    a2 = (pa[:, 0:1] * pa[:, 0:1] + pa[:, 1:2] * pa[:, 1:2]
          + pa[:, 2:3] * pa[:, 2:3])

    def rowmin(x):
        return jnp.min(x, axis=1, keepdims=True)

    def tiles(rhs):
        m2ab = jax.lax.dot_general(p, rhs[0:3, :], (((1,), (0,)), ((), ())),
                                   preferred_element_type=jnp.float32)
        d = (a2 + rhs[3:4, :]) + m2ab
        v = jax.lax.dot_general(pa, rhs, (((1,), (0,)), ((), ())),
                                preferred_element_type=jnp.float32,
                                precision=jax.lax.Precision.HIGHEST)
        return d, v

    d_gt, v_gt = tiles(rgt)
    d_pp, v_pp = tiles(rpp)

    m = rowmin(d_gt)
    val = rowmin(jnp.where(d_gt == m, v_gt, BIG))
    emd_val = jnp.sum(val + a2)

    prev = rowmin(d_pp)
    rep_val = jnp.zeros((), dtype=jnp.float32)
    for _ in range(NN_SIZE - 1):
        cur = rowmin(jnp.where(d_pp > prev, d_pp, BIG))
        vk = rowmin(jnp.where(d_pp == cur, v_pp, BIG))
        d2 = jnp.maximum(vk + a2, EPS)
        dist = jnp.sqrt(d2)
        w = jnp.exp(-d2 / (H * H))
        term = jnp.where(cur < BIG, (RADIUS - dist) * w, 0.0)
        rep_val = rep_val + jnp.sum(term)
        prev = cur

    @pl.when(jnp.logical_and(b == 0, t == 0))
    def _():
        emd_ref[...] = jnp.zeros_like(emd_ref)
        rep_ref[...] = jnp.zeros_like(rep_ref)

    emd_ref[pl.ds(b, 1), :] += emd_val.reshape(1, 1)
    rep_ref[...] += rep_val.reshape(1, 1)


@functools.partial(jax.jit, static_argnames=("interpret",))
def kernel(pred, gt, pcd_radius, interpret=False):
    ones = jnp.ones(pred.shape[:2] + (1,), dtype=pred.dtype)
    p_aug = jnp.concatenate([pred, ones], axis=2)
    g2 = jnp.sum(gt * gt, axis=2, keepdims=True)
    rhs_gt = jnp.concatenate([-2.0 * gt, g2], axis=2).transpose(0, 2, 1)
    p2 = jnp.sum(pred * pred, axis=2, keepdims=True)
    rhs_pp = jnp.concatenate([-2.0 * pred, p2], axis=2).transpose(0, 2, 1)

    emd_sums, rep_sum = pl.pallas_call(
        _loss_kernel,
        grid=(B, N // TILE),
        in_specs=[
            pl.BlockSpec((1, TILE, 4), lambda b, t: (b, t, 0)),
            pl.BlockSpec((1, 4, N), lambda b, t: (b, 0, 0)),
            pl.BlockSpec((1, 4, N), lambda b, t: (b, 0, 0)),
        ],
        out_specs=[
            pl.BlockSpec((B, 1), lambda b, t: (0, 0)),
            pl.BlockSpec((1, 1), lambda b, t: (0, 0)),
        ],
        out_shape=[
            jax.ShapeDtypeStruct((B, 1), jnp.float32),
            jax.ShapeDtypeStruct((1, 1), jnp.float32),
        ],
        interpret=interpret,
    )(p_aug, rhs_gt, rhs_pp)

    dist2_mean = emd_sums / float(N * D) / pcd_radius
    emd_loss = jnp.mean(dist2_mean) * 100.0
    uniform_loss = rep_sum[0, 0] / float(B * N * (NN_SIZE - 1))
    return (emd_loss, ALPHA * uniform_loss)


# speedup vs baseline: 1.0841x; 1.0091x over previous
import functools

import jax
import jax.numpy as jnp
from jax.experimental import pallas as pl

ALPHA = 1.0
NN_SIZE = 5
RADIUS = 0.07
H = 0.03
EPS = 1e-12

B = 4
N = 2048
D = 3
TILE = 1024
BIG = 3.0e38


def _loss_kernel(paug_ref, rgt_ref, rpp_ref, emd_ref, rep_ref):
    b = pl.program_id(0)
    t = pl.program_id(1)

    pa = paug_ref[0]
    rgt = rgt_ref[0]
    rpp = rpp_ref[0]

    p = pa[:, 0:3]
    a2 = (pa[:, 0:1] * pa[:, 0:1] + pa[:, 1:2] * pa[:, 1:2]
          + pa[:, 2:3] * pa[:, 2:3])

    def rowmin(x):
        return jnp.min(x, axis=1, keepdims=True)

    def tiles(rhs):
        m2ab = jax.lax.dot_general(p, rhs[0:3, :], (((1,), (0,)), ((), ())),
                                   preferred_element_type=jnp.float32)
        d = (a2 + rhs[3:4, :]) + m2ab
        v = jax.lax.dot_general(pa, rhs, (((1,), (0,)), ((), ())),
                                preferred_element_type=jnp.float32,
                                precision=jax.lax.Precision.HIGHEST)
        return d, v

    d_gt, v_gt = tiles(rgt)
    d_pp, v_pp = tiles(rpp)

    m = rowmin(d_gt)
    val = rowmin(jnp.where(d_gt == m, v_gt, BIG))
    emd_val = jnp.sum(val + a2)

    def chain5(dd):
        outs = [rowmin(dd)]
        for _ in range(NN_SIZE - 1):
            outs.append(rowmin(jnp.where(dd > outs[-1], dd, BIG)))
        return outs

    lft = chain5(d_pp[:, : N // 2])
    rgt5 = chain5(d_pp[:, N // 2 :])
    g = [jnp.minimum(lft[0], rgt5[0])]
    for r in range(1, NN_SIZE):
        cands = [lft[r], rgt5[r]]
        for i in range(r):
            cands.append(jnp.maximum(lft[i], rgt5[r - 1 - i]))
        acc = cands[0]
        for c in cands[1:]:
            acc = jnp.minimum(acc, c)
        g.append(acc)

    rep_val = jnp.zeros((), dtype=jnp.float32)
    for k in range(1, NN_SIZE):
        cur = g[k]
        vk = rowmin(jnp.where(d_pp == cur, v_pp, BIG))
        d2 = jnp.maximum(vk + a2, EPS)
        dist = jnp.sqrt(d2)
        w = jnp.exp(-d2 / (H * H))
        term = jnp.where(cur < BIG, (RADIUS - dist) * w, 0.0)
        rep_val = rep_val + jnp.sum(term)

    @pl.when(jnp.logical_and(b == 0, t == 0))
    def _():
        emd_ref[...] = jnp.zeros_like(emd_ref)
        rep_ref[...] = jnp.zeros_like(rep_ref)

    emd_ref[pl.ds(b, 1), :] += emd_val.reshape(1, 1)
    rep_ref[...] += rep_val.reshape(1, 1)


@functools.partial(jax.jit, static_argnames=("interpret",))
def kernel(pred, gt, pcd_radius, interpret=False):
    ones = jnp.ones(pred.shape[:2] + (1,), dtype=pred.dtype)
    p_aug = jnp.concatenate([pred, ones], axis=2)
    g2 = jnp.sum(gt * gt, axis=2, keepdims=True)
    rhs_gt = jnp.concatenate([-2.0 * gt, g2], axis=2).transpose(0, 2, 1)
    p2 = jnp.sum(pred * pred, axis=2, keepdims=True)
    rhs_pp = jnp.concatenate([-2.0 * pred, p2], axis=2).transpose(0, 2, 1)

    emd_sums, rep_sum = pl.pallas_call(
        _loss_kernel,
        grid=(B, N // TILE),
        in_specs=[
            pl.BlockSpec((1, TILE, 4), lambda b, t: (b, t, 0)),
            pl.BlockSpec((1, 4, N), lambda b, t: (b, 0, 0)),
            pl.BlockSpec((1, 4, N), lambda b, t: (b, 0, 0)),
        ],
        out_specs=[
            pl.BlockSpec((B, 1), lambda b, t: (0, 0)),
            pl.BlockSpec((1, 1), lambda b, t: (0, 0)),
        ],
        out_shape=[
            jax.ShapeDtypeStruct((B, 1), jnp.float32),
            jax.ShapeDtypeStruct((1, 1), jnp.float32),
        ],
        interpret=interpret,
    )(p_aug, rhs_gt, rhs_pp)

    dist2_mean = emd_sums / float(N * D) / pcd_radius
    emd_loss = jnp.mean(dist2_mean) * 100.0
    uniform_loss = rep_sum[0, 0] / float(B * N * (NN_SIZE - 1))
    return (emd_loss, ALPHA * uniform_loss)


# speedup vs baseline: 1.5325x; 1.4136x over previous
import functools

import jax
import jax.numpy as jnp
from jax.experimental import pallas as pl

ALPHA = 1.0
NN_SIZE = 5
RADIUS = 0.07
H = 0.03
EPS = 1e-12

B = 4
N = 2048
D = 3
TILE = 1024
BIG = 3.0e38


def _loss_kernel(paug_ref, rgt_ref, rpp_ref, emd_ref, rep_ref):
    b = pl.program_id(0)
    t = pl.program_id(1)

    pa = paug_ref[0]
    rgt = rgt_ref[0]
    rpp = rpp_ref[0]

    p = pa[:, 0:3]
    a2 = (pa[:, 0:1] * pa[:, 0:1] + pa[:, 1:2] * pa[:, 1:2]
          + pa[:, 2:3] * pa[:, 2:3])

    def rowmin(x):
        return jnp.min(x, axis=1, keepdims=True)

    def tiles(rhs):
        m2ab = jax.lax.dot_general(p, rhs[0:3, :], (((1,), (0,)), ((), ())),
                                   preferred_element_type=jnp.float32)
        d = (a2 + rhs[3:4, :]) + m2ab
        v = jax.lax.dot_general(pa[:, 3:14], rhs[4:15, :],
                                (((1,), (0,)), ((), ())),
                                preferred_element_type=jnp.float32)
        return d, v

    d_gt, v_gt = tiles(rgt)
    d_pp, v_pp = tiles(rpp)

    m = rowmin(d_gt)
    val = rowmin(jnp.where(d_gt == m, v_gt, BIG))
    emd_val = jnp.sum(val + a2)

    def chain5(dd):
        outs = [rowmin(dd)]
        for _ in range(NN_SIZE - 1):
            outs.append(rowmin(jnp.where(dd > outs[-1], dd, BIG)))
        return outs

    lft = chain5(d_pp[:, : N // 2])
    rgt5 = chain5(d_pp[:, N // 2 :])
    g = [jnp.minimum(lft[0], rgt5[0])]
    for r in range(1, NN_SIZE):
        cands = [lft[r], rgt5[r]]
        for i in range(r):
            cands.append(jnp.maximum(lft[i], rgt5[r - 1 - i]))
        acc = cands[0]
        for c in cands[1:]:
            acc = jnp.minimum(acc, c)
        g.append(acc)

    rep_val = jnp.zeros((), dtype=jnp.float32)
    for k in range(1, NN_SIZE):
        cur = g[k]
        vk = rowmin(jnp.where(d_pp == cur, v_pp, BIG))
        d2 = jnp.maximum(vk + a2, EPS)
        dist = jnp.sqrt(d2)
        w = jnp.exp(-d2 / (H * H))
        term = jnp.where(cur < BIG, (RADIUS - dist) * w, 0.0)
        rep_val = rep_val + jnp.sum(term)

    @pl.when(jnp.logical_and(b == 0, t == 0))
    def _():
        emd_ref[...] = jnp.zeros_like(emd_ref)
        rep_ref[...] = jnp.zeros_like(rep_ref)

    emd_ref[pl.ds(b, 1), :] += emd_val.reshape(1, 1)
    rep_ref[...] += rep_val.reshape(1, 1)


@functools.partial(jax.jit, static_argnames=("interpret",))
def kernel(pred, gt, pcd_radius, interpret=False):
    def split(x):
        hi = x.astype(jnp.bfloat16).astype(jnp.float32)
        return hi, x - hi

    ones = jnp.ones(pred.shape[:2] + (1,), dtype=pred.dtype)
    p_hi, p_lo = split(pred)
    p_aug = jnp.concatenate([pred, p_hi, p_hi, p_lo, ones, ones], axis=2)

    def make_rhs(q):
        g = -2.0 * q
        g_hi, g_lo = split(g)
        q2 = jnp.sum(q * q, axis=2, keepdims=True)
        q2_hi, q2_lo = split(q2)
        return jnp.concatenate([g, q2, g_hi, g_lo, g_hi, q2_hi, q2_lo],
                               axis=2).transpose(0, 2, 1)

    rhs_gt = make_rhs(gt)
    rhs_pp = make_rhs(pred)

    emd_sums, rep_sum = pl.pallas_call(
        _loss_kernel,
        grid=(B, N // TILE),
        in_specs=[
            pl.BlockSpec((1, TILE, 14), lambda b, t: (b, t, 0)),
            pl.BlockSpec((1, 15, N), lambda b, t: (b, 0, 0)),
            pl.BlockSpec((1, 15, N), lambda b, t: (b, 0, 0)),
        ],
        out_specs=[
            pl.BlockSpec((B, 1), lambda b, t: (0, 0)),
            pl.BlockSpec((1, 1), lambda b, t: (0, 0)),
        ],
        out_shape=[
            jax.ShapeDtypeStruct((B, 1), jnp.float32),
            jax.ShapeDtypeStruct((1, 1), jnp.float32),
        ],
        interpret=interpret,
    )(p_aug, rhs_gt, rhs_pp)

    dist2_mean = emd_sums / float(N * D) / pcd_radius
    emd_loss = jnp.mean(dist2_mean) * 100.0
    uniform_loss = rep_sum[0, 0] / float(B * N * (NN_SIZE - 1))
    return (emd_loss, ALPHA * uniform_loss)
